# baseline (speedup 1.0000x reference)
import functools

import jax
import jax.numpy as jnp
from jax import lax
from jax.experimental import pallas as pl
from jax.experimental.pallas import tpu as pltpu
from jax.experimental.pallas import tpu_sc as plsc

NC = 2
NS = 16
NW = NC * NS
L = 16


def _linear_parts_kernel(h_ref, wpad_ref, wt_ref, scal_ref, a_ref, c_ref, wb_ref):
    ac = lax.dot_general(
        wpad_ref[...], h_ref[...],
        dimension_numbers=(((1,), (1,)), ((), ())),
        preferred_element_type=jnp.float32,
    )
    a_ref[...] = ac[0:1]
    c_ref[...] = ac[1:2]
    wb = wt_ref[...] * scal_ref[0, 0] + scal_ref[0, 1]
    wb_ref[...] = wb.reshape(wb_ref.shape)


def _fill_kernel(o_ref):
    o_ref[...] = jnp.full_like(o_ref, -jnp.inf)


def _sc_mesh():
    return plsc.VectorSubcoreMesh(
        core_axis_name="c", subcore_axis_name="s",
        num_cores=NC, num_subcores=NS,
    )


def _make_edge_vals(n, e, ept):
    @functools.partial(
        pl.kernel, mesh=_sc_mesh(),
        out_type=(
            jax.ShapeDtypeStruct((e,), jnp.int32),
            jax.ShapeDtypeStruct((e,), jnp.float32),
        ),
        compiler_params=pltpu.CompilerParams(needs_layout_passes=False),
        scratch_types=[
            pltpu.VMEM((1, n), jnp.float32),
            pltpu.VMEM((1, n), jnp.float32),
            pltpu.VMEM((ept,), jnp.int32),
            pltpu.VMEM((ept,), jnp.int32),
            pltpu.VMEM((ept,), jnp.float32),
            pltpu.VMEM((ept,), jnp.int32),
            pltpu.VMEM((ept,), jnp.float32),
        ],
    )
    def edge_vals(a_hbm, c_hbm, wb_hbm, src_hbm, dst_hbm, idx_hbm, valo_hbm,
                  a_v, c_v, src_v, dst_v, wb_v, idx_v, val_v):
        wid = lax.axis_index("s") * NC + lax.axis_index("c")
        base = wid * ept
        pltpu.sync_copy(a_hbm, a_v)
        pltpu.sync_copy(c_hbm, c_v)
        pltpu.sync_copy(src_hbm.at[pl.ds(base, ept)], src_v)
        pltpu.sync_copy(dst_hbm.at[pl.ds(base, ept)], dst_v)
        pltpu.sync_copy(wb_hbm.at[pl.ds(base, ept)], wb_v)

        def group(g, _):
            o = g * L
            s = src_v[pl.ds(o, L)]
            t = dst_v[pl.ds(o, L)]
            va = plsc.load_gather(a_v.at[0], [s])
            vc = plsc.load_gather(c_v.at[0], [t])
            idx_v[pl.ds(o, L)] = s * n + t
            val_v[pl.ds(o, L)] = va + vc + wb_v[pl.ds(o, L)]
            return 0

        lax.fori_loop(0, ept // L, group, 0)
        pltpu.sync_copy(idx_v, idx_hbm.at[pl.ds(base, ept)])
        pltpu.sync_copy(val_v, valo_hbm.at[pl.ds(base, ept)])

    return edge_vals


def _make_scatter(ept, hhalf):
    @functools.partial(
        pl.kernel, mesh=_sc_mesh(), out_type=(),
        compiler_params=pltpu.CompilerParams(needs_layout_passes=False),
        scratch_types=[
            pltpu.VMEM((ept,), jnp.int32),
            pltpu.VMEM((ept,), jnp.float32),
            pltpu.VMEM((ept,), jnp.int32),
            pltpu.VMEM((ept,), jnp.float32),
            pltpu.SemaphoreType.DMA,
        ],
    )
    def scatter(scores_ref, idx_hbm, val_hbm, idx_v, val_v, idx2_v, val2_v, sem):
        sid = lax.axis_index("s")
        cid = lax.axis_index("c")
        b0 = (sid * NC) * ept
        b1 = (sid * NC + 1) * ept
        pltpu.sync_copy(idx_hbm.at[pl.ds(b0, ept)], idx_v)
        pltpu.sync_copy(val_hbm.at[pl.ds(b0, ept)], val_v)
        pltpu.sync_copy(idx_hbm.at[pl.ds(b1, ept)], idx2_v)
        pltpu.sync_copy(val_hbm.at[pl.ds(b1, ept)], val2_v)

        half = jnp.int32(hhalf)

        def mask(ref):
            def body(g, _):
                o = g * L
                v = ref[pl.ds(o, L)]
                keep = jnp.where(cid == 0, v < half, v >= half)
                ref[pl.ds(o, L)] = jnp.where(keep, v, -1)
                return 0
            lax.fori_loop(0, ept // L, body, 0)

        mask(idx_v)
        mask(idx2_v)
        cp1 = pltpu.async_copy(
            val_v, scores_ref.at[plsc.Indices(idx_v, ignored_value=-1)], sem)
        cp2 = pltpu.async_copy(
            val2_v, scores_ref.at[plsc.Indices(idx2_v, ignored_value=-1)], sem)
        cp1.wait()
        cp2.wait()

    return scatter


def kernel(h, sources, dists, weights, W, b):
    n, d = h.shape
    e = sources.shape[0]
    assert e % (NW * L) == 0
    ept = e // NW

    wpad = W[: 2 * d, 0].reshape(2, d)
    scal = jnp.reshape(jnp.stack([W[2 * d, 0], b[0]]), (1, 2))
    srci = sources.astype(jnp.int32)
    dsti = dists.astype(jnp.int32)
    wt2 = weights.reshape(1, e)

    a_part, c_part, wb = pl.pallas_call(
        _linear_parts_kernel,
        out_shape=(
            jax.ShapeDtypeStruct((1, n), jnp.float32),
            jax.ShapeDtypeStruct((1, n), jnp.float32),
            jax.ShapeDtypeStruct((e // 128, 128), jnp.float32),
        ),
        in_specs=[
            pl.BlockSpec(memory_space=pltpu.VMEM),
            pl.BlockSpec(memory_space=pltpu.VMEM),
            pl.BlockSpec(memory_space=pltpu.VMEM),
            pl.BlockSpec(memory_space=pltpu.SMEM),
        ],
        out_specs=(
            pl.BlockSpec(memory_space=pltpu.VMEM),
            pl.BlockSpec(memory_space=pltpu.VMEM),
            pl.BlockSpec(memory_space=pltpu.VMEM),
        ),
    )(h, wpad, wt2, scal)
    wb_flat = wb.reshape(e)

    idx_all, val_all = _make_edge_vals(n, e, ept)(
        a_part, c_part, wb_flat, srci, dsti)

    blk = 2 ** 21
    scores0 = pl.pallas_call(
        _fill_kernel,
        grid=(pl.cdiv(n * n, blk),),
        out_shape=jax.ShapeDtypeStruct((n * n,), jnp.float32),
        out_specs=pl.BlockSpec((blk,), lambda i: (i,)),
    )()

    scores_ref = jax.new_ref(scores0)
    _make_scatter(ept, n * n // 2)(scores_ref, idx_all, val_all)
    return scores_ref[...].reshape(n, n)


# speedup vs baseline: 1.0106x; 1.0106x over previous
import functools

import jax
import jax.numpy as jnp
from jax import lax
from jax.experimental import pallas as pl
from jax.experimental.pallas import tpu as pltpu
from jax.experimental.pallas import tpu_sc as plsc

NC = 2
NS = 16
NW = NC * NS
L = 16


def _linear_parts_kernel(h_ref, wpad_ref, wt_ref, scal_ref, a_ref, c_ref, wb_ref):
    ac = lax.dot_general(
        wpad_ref[...], h_ref[...],
        dimension_numbers=(((1,), (1,)), ((), ())),
        preferred_element_type=jnp.float32,
    )
    a_ref[...] = ac[0:1]
    c_ref[...] = ac[1:2]
    wb = wt_ref[...] * scal_ref[0, 0] + scal_ref[0, 1]
    wb_ref[...] = wb.reshape(wb_ref.shape)


def _fill_kernel(o_ref):
    o_ref[...] = jnp.full_like(o_ref, -jnp.inf)


def _sc_mesh():
    return plsc.VectorSubcoreMesh(
        core_axis_name="c", subcore_axis_name="s",
        num_cores=NC, num_subcores=NS,
    )


def _make_edge_vals(n, e, ept):
    @functools.partial(
        pl.kernel, mesh=_sc_mesh(),
        out_type=(
            jax.ShapeDtypeStruct((e,), jnp.int32),
            jax.ShapeDtypeStruct((e,), jnp.float32),
        ),
        compiler_params=pltpu.CompilerParams(needs_layout_passes=False),
        scratch_types=[
            pltpu.VMEM((1, n), jnp.float32),
            pltpu.VMEM((1, n), jnp.float32),
            pltpu.VMEM((ept,), jnp.int32),
            pltpu.VMEM((ept,), jnp.int32),
            pltpu.VMEM((ept,), jnp.float32),
            pltpu.VMEM((ept,), jnp.int32),
            pltpu.VMEM((ept,), jnp.float32),
        ],
    )
    def edge_vals(a_hbm, c_hbm, wb_hbm, src_hbm, dst_hbm, idx_hbm, valo_hbm,
                  a_v, c_v, src_v, dst_v, wb_v, idx_v, val_v):
        wid = lax.axis_index("s") * NC + lax.axis_index("c")
        base = wid * ept
        pltpu.sync_copy(a_hbm, a_v)
        pltpu.sync_copy(c_hbm, c_v)
        pltpu.sync_copy(src_hbm.at[pl.ds(base, ept)], src_v)
        pltpu.sync_copy(dst_hbm.at[pl.ds(base, ept)], dst_v)
        pltpu.sync_copy(wb_hbm.at[pl.ds(base, ept)], wb_v)

        def group(g, _):
            o = g * L
            s = src_v[pl.ds(o, L)]
            t = dst_v[pl.ds(o, L)]
            va = plsc.load_gather(a_v.at[0], [s])
            vc = plsc.load_gather(c_v.at[0], [t])
            idx_v[pl.ds(o, L)] = s * n + t
            val_v[pl.ds(o, L)] = va + vc + wb_v[pl.ds(o, L)]
            return 0

        lax.fori_loop(0, ept // L, group, 0)
        pltpu.sync_copy(idx_v, idx_hbm.at[pl.ds(base, ept)])
        pltpu.sync_copy(val_v, valo_hbm.at[pl.ds(base, ept)])

    return edge_vals


def _make_scatter(ept):
    @functools.partial(
        pl.kernel, mesh=_sc_mesh(), out_type=(),
        compiler_params=pltpu.CompilerParams(needs_layout_passes=False),
        scratch_types=[
            pltpu.VMEM((ept,), jnp.int32),
            pltpu.VMEM((ept,), jnp.float32),
            pltpu.SemaphoreType.DMA,
        ],
    )
    def scatter(scores_ref, idx_hbm, val_hbm, idx_v, val_v, sem):
        wid = lax.axis_index("s") * NC + lax.axis_index("c")
        base = wid * ept
        pltpu.sync_copy(idx_hbm.at[pl.ds(base, ept)], idx_v)
        pltpu.sync_copy(val_hbm.at[pl.ds(base, ept)], val_v)
        pltpu.async_copy(val_v, scores_ref.at[idx_v], sem).wait()

    return scatter


def kernel(h, sources, dists, weights, W, b):
    n, d = h.shape
    e = sources.shape[0]
    assert e % (NW * L) == 0
    ept = e // NW

    wpad = W[: 2 * d, 0].reshape(2, d)
    scal = jnp.reshape(jnp.stack([W[2 * d, 0], b[0]]), (1, 2))
    srci = sources.astype(jnp.int32)
    dsti = dists.astype(jnp.int32)
    wt2 = weights.reshape(1, e)

    a_part, c_part, wb = pl.pallas_call(
        _linear_parts_kernel,
        out_shape=(
            jax.ShapeDtypeStruct((1, n), jnp.float32),
            jax.ShapeDtypeStruct((1, n), jnp.float32),
            jax.ShapeDtypeStruct((e // 128, 128), jnp.float32),
        ),
        in_specs=[
            pl.BlockSpec(memory_space=pltpu.VMEM),
            pl.BlockSpec(memory_space=pltpu.VMEM),
            pl.BlockSpec(memory_space=pltpu.VMEM),
            pl.BlockSpec(memory_space=pltpu.SMEM),
        ],
        out_specs=(
            pl.BlockSpec(memory_space=pltpu.VMEM),
            pl.BlockSpec(memory_space=pltpu.VMEM),
            pl.BlockSpec(memory_space=pltpu.VMEM),
        ),
    )(h, wpad, wt2, scal)
    wb_flat = wb.reshape(e)

    idx_all, val_all = _make_edge_vals(n, e, ept)(
        a_part, c_part, wb_flat, srci, dsti)

    blk = 2 ** 21
    scores0 = pl.pallas_call(
        _fill_kernel,
        grid=(pl.cdiv(n * n, blk),),
        out_shape=jax.ShapeDtypeStruct((n * n,), jnp.float32),
        out_specs=pl.BlockSpec((blk,), lambda i: (i,)),
    )()

    scores_ref = jax.new_ref(scores0)
    _make_scatter(ept)(scores_ref, idx_all, val_all)
    return scores_ref[...].reshape(n, n)
